# baseline (speedup 1.0000x reference)
import jax
import jax.numpy as jnp
from jax import lax
from jax.experimental import pallas as pl
from jax.experimental.pallas import tpu as pltpu
from jax.experimental.pallas import tpu_sc as plsc

B = 1024
K = 5
SYN = 64
SEM = 32
LANES = 128
HV_PER_SUP = LANES // SEM
CAT_PER_SUP = LANES // SYN
NC = 2
NS = 16
NW = NC * NS
CAT_PW = B // NW
HV_TOTAL = 2 * B * K
HV_CHUNK = 80
HV_CHUNKS_PW = HV_TOTAL // (NW * HV_CHUNK)
HV_PW = HV_CHUNK * HV_CHUNKS_PW


def _compact_body(in_ref, out_ref):
    x = in_ref[...]
    d = x.shape[0]
    r = LANES // d
    slabs = []
    for s in range(x.shape[1] // (r * LANES)):
        v = jnp.concatenate(
            [x[:, (s * r + m) * LANES:(s * r + m + 1) * LANES]
             for m in range(r)], axis=0)
        slabs.append(jnp.transpose(v))
    out_ref[...] = jnp.concatenate(slabs, axis=0)


def _compact(table_t, blk_cols):
    d, v = table_t.shape
    blk_rows = d * blk_cols // LANES
    n_blocks = -(-v // blk_cols)
    out_rows = n_blocks * blk_rows
    return pl.pallas_call(
        _compact_body,
        grid=(n_blocks,),
        in_specs=[pl.BlockSpec((d, blk_cols), lambda c: (0, c))],
        out_specs=pl.BlockSpec((blk_rows, LANES), lambda c: (c, 0)),
        out_shape=jax.ShapeDtypeStruct((out_rows, LANES), jnp.float32),
    )(table_t)


def _sc_body(cat_ix_hbm, hv_ix_hbm, cat_sup_hbm, hv_sup_hbm,
             cat_out_hbm, hv_out_hbm,
             cat_idx_v, hv_idx_v, cat_rows_v, hv_rows_v, sem):
    wid = lax.axis_index("s") * NC + lax.axis_index("c")
    pltpu.sync_copy(cat_ix_hbm.at[pl.ds(wid * CAT_PW, CAT_PW)], cat_idx_v)
    pltpu.sync_copy(hv_ix_hbm.at[pl.ds(wid * HV_CHUNKS_PW, HV_CHUNKS_PW)],
                    hv_idx_v)
    copies = [pltpu.async_copy(cat_sup_hbm.at[cat_idx_v], cat_rows_v, sem)]
    for j in range(HV_CHUNKS_PW):
        copies.append(pltpu.async_copy(
            hv_sup_hbm.at[hv_idx_v.at[j]],
            hv_rows_v.at[pl.ds(j * HV_CHUNK, HV_CHUNK)], sem))
    for c in copies:
        c.wait()
    pltpu.sync_copy(cat_rows_v, cat_out_hbm.at[pl.ds(wid * CAT_PW, CAT_PW)])
    pltpu.sync_copy(hv_rows_v, hv_out_hbm.at[pl.ds(wid * HV_PW, HV_PW)])


def _sc_gather(cat_sup_ix, hv_sup_ix, cat_sup, hv_sup):
    mesh = plsc.VectorSubcoreMesh(core_axis_name="c", subcore_axis_name="s")
    return pl.kernel(
        _sc_body,
        mesh=mesh,
        out_type=(
            jax.ShapeDtypeStruct((B, LANES), jnp.float32),
            jax.ShapeDtypeStruct((HV_TOTAL, LANES), jnp.float32),
        ),
        scratch_types=[
            pltpu.VMEM((CAT_PW,), jnp.int32),
            pltpu.VMEM((HV_CHUNKS_PW, HV_CHUNK), jnp.int32),
            pltpu.VMEM((CAT_PW, LANES), jnp.float32),
            pltpu.VMEM((HV_PW, LANES), jnp.float32),
            pltpu.SemaphoreType.DMA,
        ],
    )(cat_sup_ix, hv_sup_ix, cat_sup, hv_sup)


def _select_sub(rows, sub, width):
    out = jnp.where(sub == 0, rows[:, 0:width], 0.0)
    for p in range(1, LANES // width):
        out = out + jnp.where(sub == p, rows[:, p * width:(p + 1) * width],
                              0.0)
    return out


def _dedup_weighted_sum(ix, hv, base):
    sub = jnp.bitwise_and(jnp.right_shift(ix, 7), HV_PER_SUP - 1)
    acc = _select_sub(hv[base], sub[:, 0:1], SEM)
    for j in range(1, K):
        dup = (ix[:, 0:1] == ix[:, j:j + 1]).astype(jnp.float32)
        for i in range(1, j):
            dup = jnp.maximum(
                dup, (ix[:, i:i + 1] == ix[:, j:j + 1]).astype(jnp.float32))
        acc = acc + (1.0 - dup) * _select_sub(hv[base + j], sub[:, j:j + 1],
                                              SEM)
    return acc


def _tc_body(cat_rows_ref, hv_rows_ref, cat_ix_ref, hvb_ix_ref, hvf_ix_ref,
             hvb_top_ref, hvf_top_ref, d_onehot_ref,
             w1t_ref, b1_ref, w2t_ref, b2_ref, out_ref):
    hv = hv_rows_ref[...]
    hvb_e = _dedup_weighted_sum(hvb_ix_ref[...], hv, 0) + hvb_top_ref[...]
    hvf_e = _dedup_weighted_sum(hvf_ix_ref[...], hv, K) + hvf_top_ref[...]
    cat_sub = jnp.bitwise_and(jnp.right_shift(cat_ix_ref[...], 7),
                              CAT_PER_SUP - 1)
    cat_e = _select_sub(cat_rows_ref[...], cat_sub, SYN)
    x = jnp.concatenate([cat_e, hvb_e, hvf_e, d_onehot_ref[...]], axis=1)
    h = jnp.maximum(
        jnp.dot(x, w1t_ref[...], preferred_element_type=jnp.float32)
        + b1_ref[...], 0.0)
    o = jnp.dot(h, w2t_ref[...], preferred_element_type=jnp.float32)
    o = o + b2_ref[...]
    m = jnp.max(o, axis=1, keepdims=True)
    s = o - m
    out_ref[...] = s - jnp.log(jnp.sum(jnp.exp(s), axis=1, keepdims=True))


def _tc_mlp(cat_rows, hv_rows, cat_ix, hvb_ix, hvf_ix, hvb_top, hvf_top,
            d_onehot, w1t, b1, w2t, b2):
    out_dim = w2t.shape[1]
    nb = 4
    bb = B // nb
    full = lambda shape: pl.BlockSpec(shape, lambda g: tuple(0 for _ in shape))
    row = lambda cols: pl.BlockSpec((bb, cols), lambda g: (g, 0))
    return pl.pallas_call(
        _tc_body,
        grid=(nb,),
        in_specs=[
            row(LANES),
            pl.BlockSpec((2 * K, bb, LANES), lambda g: (0, g, 0)),
            row(1), row(K), row(K), row(SEM), row(SEM), row(7),
            full(w1t.shape), full(b1.shape), full(w2t.shape), full(b2.shape),
        ],
        out_specs=row(out_dim),
        out_shape=jax.ShapeDtypeStruct((B, out_dim), jnp.float32),
    )(cat_rows, hv_rows, cat_ix, hvb_ix, hvf_ix, hvb_top, hvf_top,
      d_onehot, w1t, b1, w2t, b2)


def kernel(d_onehot, cat_b_ix, hvb_ix, hvf_ix, hvb_top, hvf_top, use_gpu,
           cat_emb, hvec_emb, fc1_w, fc1_b, fc2_w, fc2_b):
    cat_ix = cat_b_ix.astype(jnp.int32)
    hvb_i = hvb_ix.astype(jnp.int32)
    hvf_i = hvf_ix.astype(jnp.int32)
    hv_flat = jnp.concatenate([hvb_i.T.reshape(-1), hvf_i.T.reshape(-1)])
    hv_sup_ix = (
        ((hv_flat >> 9) << 7) + (hv_flat & 127)
    ).reshape(HV_TOTAL // HV_CHUNK, HV_CHUNK)
    cat_sup_ix = ((cat_ix >> 8) << 7) + (cat_ix & 127)
    cat_sup = _compact(cat_emb.T, 4096)
    hv_sup = _compact(hvec_emb.T, 8192)
    cat_rows, hv_rows = _sc_gather(cat_sup_ix, hv_sup_ix, cat_sup, hv_sup)
    return _tc_mlp(cat_rows, hv_rows.reshape(2 * K, B, LANES),
                   cat_ix.reshape(B, 1), hvb_i, hvf_i,
                   hvb_top, hvf_top, d_onehot,
                   fc1_w.T, fc1_b.reshape(1, -1),
                   fc2_w.T, fc2_b.reshape(1, -1))


# speedup vs baseline: 1.0575x; 1.0575x over previous
import jax
import jax.numpy as jnp
from jax import lax
from jax.experimental import pallas as pl
from jax.experimental.pallas import tpu as pltpu
from jax.experimental.pallas import tpu_sc as plsc

B = 1024
K = 5
SYN = 64
SEM = 32
LANES = 128
HV_PER_SUP = LANES // SEM
CAT_PER_SUP = LANES // SYN
NC = 2
NS = 16
NW = NC * NS
CAT_PW = B // NW
HV_TOTAL = 2 * B * K
HV_CHUNK = 80
HV_CHUNKS_PW = HV_TOTAL // (NW * HV_CHUNK)
HV_PW = HV_CHUNK * HV_CHUNKS_PW


def _compact_body(in_ref, out_ref):
    x = in_ref[...]
    d = x.shape[0]
    r = LANES // d
    slabs = []
    for s in range(x.shape[1] // (r * LANES)):
        v = jnp.concatenate(
            [x[:, (s * r + m) * LANES:(s * r + m + 1) * LANES]
             for m in range(r)], axis=0)
        slabs.append(jnp.transpose(v))
    out_ref[...] = jnp.concatenate(slabs, axis=0)


def _compact(table_t, blk_cols):
    d, v = table_t.shape
    blk_rows = d * blk_cols // LANES
    n_blocks = -(-v // blk_cols)
    out_rows = n_blocks * blk_rows
    return pl.pallas_call(
        _compact_body,
        grid=(n_blocks,),
        in_specs=[pl.BlockSpec((d, blk_cols), lambda c: (0, c))],
        out_specs=pl.BlockSpec((blk_rows, LANES), lambda c: (c, 0)),
        out_shape=jax.ShapeDtypeStruct((out_rows, LANES), jnp.float32),
    )(table_t)


def _sc_body(cat_ix_hbm, hv_ix_hbm, cat_sup_hbm, hv_sup_hbm,
             cat_out_hbm, hv_out_hbm,
             cat_idx_v, hv_idx_v, cat_rows_v, hv_rows_v, sem):
    wid = lax.axis_index("s") * NC + lax.axis_index("c")
    pltpu.sync_copy(cat_ix_hbm.at[pl.ds(wid * CAT_PW, CAT_PW)], cat_idx_v)
    pltpu.sync_copy(hv_ix_hbm.at[pl.ds(wid * HV_CHUNKS_PW, HV_CHUNKS_PW)],
                    hv_idx_v)
    copies = [pltpu.async_copy(cat_sup_hbm.at[cat_idx_v], cat_rows_v, sem)]
    for j in range(HV_CHUNKS_PW):
        copies.append(pltpu.async_copy(
            hv_sup_hbm.at[hv_idx_v.at[j]],
            hv_rows_v.at[pl.ds(j * HV_CHUNK, HV_CHUNK)], sem))
    for c in copies:
        c.wait()
    pltpu.sync_copy(cat_rows_v, cat_out_hbm.at[pl.ds(wid * CAT_PW, CAT_PW)])
    pltpu.sync_copy(hv_rows_v, hv_out_hbm.at[pl.ds(wid * HV_PW, HV_PW)])


def _sc_gather(cat_sup_ix, hv_sup_ix, cat_sup, hv_sup):
    mesh = plsc.VectorSubcoreMesh(core_axis_name="c", subcore_axis_name="s")
    return pl.kernel(
        _sc_body,
        mesh=mesh,
        out_type=(
            jax.ShapeDtypeStruct((B, LANES), jnp.float32),
            jax.ShapeDtypeStruct((HV_TOTAL, LANES), jnp.float32),
        ),
        scratch_types=[
            pltpu.VMEM((CAT_PW,), jnp.int32),
            pltpu.VMEM((HV_CHUNKS_PW, HV_CHUNK), jnp.int32),
            pltpu.VMEM((CAT_PW, LANES), jnp.float32),
            pltpu.VMEM((HV_PW, LANES), jnp.float32),
            pltpu.SemaphoreType.DMA,
        ],
    )(cat_sup_ix, hv_sup_ix, cat_sup, hv_sup)


def _select_sub(rows, sub, width):
    out = jnp.where(sub == 0, rows[:, 0:width], 0.0)
    for p in range(1, LANES // width):
        out = out + jnp.where(sub == p, rows[:, p * width:(p + 1) * width],
                              0.0)
    return out


def _dedup_weighted_sum(ix, hv, base):
    sub = jnp.bitwise_and(jnp.right_shift(ix, 7), HV_PER_SUP - 1)
    acc = _select_sub(hv[base], sub[:, 0:1], SEM)
    for j in range(1, K):
        dup = (ix[:, 0:1] == ix[:, j:j + 1]).astype(jnp.float32)
        for i in range(1, j):
            dup = jnp.maximum(
                dup, (ix[:, i:i + 1] == ix[:, j:j + 1]).astype(jnp.float32))
        acc = acc + (1.0 - dup) * _select_sub(hv[base + j], sub[:, j:j + 1],
                                              SEM)
    return acc


def _tc_body(cat_rows_ref, hv_rows_ref, cat_ix_ref, hvb_ix_ref, hvf_ix_ref,
             hvb_top_ref, hvf_top_ref, d_onehot_ref,
             w1t_ref, b1_ref, w2t_ref, b2_ref, out_ref):
    hv = hv_rows_ref[...]
    hvb_e = _dedup_weighted_sum(hvb_ix_ref[...], hv, 0) + hvb_top_ref[...]
    hvf_e = _dedup_weighted_sum(hvf_ix_ref[...], hv, K) + hvf_top_ref[...]
    cat_sub = jnp.bitwise_and(jnp.right_shift(cat_ix_ref[...], 7),
                              CAT_PER_SUP - 1)
    cat_e = _select_sub(cat_rows_ref[...], cat_sub, SYN)
    x = jnp.concatenate([cat_e, hvb_e, hvf_e, d_onehot_ref[...]], axis=1)
    h = jnp.maximum(
        jnp.dot(x, w1t_ref[...], preferred_element_type=jnp.float32)
        + b1_ref[...], 0.0)
    o = jnp.dot(h, w2t_ref[...], preferred_element_type=jnp.float32)
    o = o + b2_ref[...]
    m = jnp.max(o, axis=1, keepdims=True)
    s = o - m
    out_ref[...] = s - jnp.log(jnp.sum(jnp.exp(s), axis=1, keepdims=True))


def _tc_mlp(cat_rows, hv_rows, cat_ix, hvb_ix, hvf_ix, hvb_top, hvf_top,
            d_onehot, w1t, b1, w2t, b2):
    out_dim = w2t.shape[1]
    nb = 2
    bb = B // nb
    full = lambda shape: pl.BlockSpec(shape, lambda g: tuple(0 for _ in shape))
    row = lambda cols: pl.BlockSpec((bb, cols), lambda g: (g, 0))
    return pl.pallas_call(
        _tc_body,
        grid=(nb,),
        in_specs=[
            row(LANES),
            pl.BlockSpec((2 * K, bb, LANES), lambda g: (0, g, 0)),
            row(1), row(K), row(K), row(SEM), row(SEM), row(7),
            full(w1t.shape), full(b1.shape), full(w2t.shape), full(b2.shape),
        ],
        out_specs=row(out_dim),
        out_shape=jax.ShapeDtypeStruct((B, out_dim), jnp.float32),
    )(cat_rows, hv_rows, cat_ix, hvb_ix, hvf_ix, hvb_top, hvf_top,
      d_onehot, w1t, b1, w2t, b2)


def kernel(d_onehot, cat_b_ix, hvb_ix, hvf_ix, hvb_top, hvf_top, use_gpu,
           cat_emb, hvec_emb, fc1_w, fc1_b, fc2_w, fc2_b):
    cat_ix = cat_b_ix.astype(jnp.int32)
    hvb_i = hvb_ix.astype(jnp.int32)
    hvf_i = hvf_ix.astype(jnp.int32)
    hv_flat = jnp.concatenate([hvb_i.T.reshape(-1), hvf_i.T.reshape(-1)])
    hv_sup_ix = (
        ((hv_flat >> 9) << 7) + (hv_flat & 127)
    ).reshape(HV_TOTAL // HV_CHUNK, HV_CHUNK)
    cat_sup_ix = ((cat_ix >> 8) << 7) + (cat_ix & 127)
    cat_sup = _compact(cat_emb.T, 4096)
    hv_sup = _compact(hvec_emb.T, 8192)
    cat_rows, hv_rows = _sc_gather(cat_sup_ix, hv_sup_ix, cat_sup, hv_sup)
    return _tc_mlp(cat_rows, hv_rows.reshape(2 * K, B, LANES),
                   cat_ix.reshape(B, 1), hvb_i, hvf_i,
                   hvb_top, hvf_top, d_onehot,
                   fc1_w.T, fc1_b.reshape(1, -1),
                   fc2_w.T, fc2_b.reshape(1, -1))


# speedup vs baseline: 1.1256x; 1.0644x over previous
import jax
import jax.numpy as jnp
from jax import lax
from jax.experimental import pallas as pl
from jax.experimental.pallas import tpu as pltpu
from jax.experimental.pallas import tpu_sc as plsc

B = 1024
K = 5
SYN = 64
SEM = 32
LANES = 128
HV_PER_SUP = LANES // SEM
CAT_PER_SUP = LANES // SYN
NC = 2
NS = 16
NW = NC * NS
CAT_PW = B // NW
HV_TOTAL = 2 * B * K
HV_CHUNK = 80
HV_CHUNKS_PW = HV_TOTAL // (NW * HV_CHUNK)
HV_PW = HV_CHUNK * HV_CHUNKS_PW


def _compact_body(in_ref, out_ref):
    x = in_ref[...]
    d = x.shape[0]
    r = LANES // d
    slabs = []
    for s in range(x.shape[1] // (r * LANES)):
        v = jnp.concatenate(
            [x[:, (s * r + m) * LANES:(s * r + m + 1) * LANES]
             for m in range(r)], axis=0)
        slabs.append(jnp.transpose(v))
    out_ref[...] = jnp.concatenate(slabs, axis=0)


def _compact(table_t, blk_cols):
    d, v = table_t.shape
    blk_rows = d * blk_cols // LANES
    n_blocks = -(-v // blk_cols)
    out_rows = n_blocks * blk_rows
    return pl.pallas_call(
        _compact_body,
        grid=(n_blocks,),
        in_specs=[pl.BlockSpec((d, blk_cols), lambda c: (0, c))],
        out_specs=pl.BlockSpec((blk_rows, LANES), lambda c: (c, 0)),
        out_shape=jax.ShapeDtypeStruct((out_rows, LANES), jnp.float32),
    )(table_t)


def _sc_body(cat_ix_hbm, hv_ix_hbm, cat_sup_hbm, hv_sup_hbm,
             cat_out_hbm, hv_out_hbm,
             cat_idx_v, hv_idx_v, cat_rows_v, hv_rows_v, sem):
    wid = lax.axis_index("s") * NC + lax.axis_index("c")
    pltpu.sync_copy(cat_ix_hbm.at[pl.ds(wid * CAT_PW, CAT_PW)], cat_idx_v)
    pltpu.sync_copy(hv_ix_hbm.at[pl.ds(wid * HV_CHUNKS_PW, HV_CHUNKS_PW)],
                    hv_idx_v)
    copies = [pltpu.async_copy(cat_sup_hbm.at[cat_idx_v], cat_rows_v, sem)]
    for j in range(HV_CHUNKS_PW):
        copies.append(pltpu.async_copy(
            hv_sup_hbm.at[hv_idx_v.at[j]],
            hv_rows_v.at[pl.ds(j * HV_CHUNK, HV_CHUNK)], sem))
    for c in copies:
        c.wait()
    pltpu.sync_copy(cat_rows_v, cat_out_hbm.at[pl.ds(wid * CAT_PW, CAT_PW)])
    pltpu.sync_copy(hv_rows_v, hv_out_hbm.at[pl.ds(wid * HV_PW, HV_PW)])


def _sc_gather(cat_sup_ix, hv_sup_ix, cat_sup, hv_sup):
    mesh = plsc.VectorSubcoreMesh(core_axis_name="c", subcore_axis_name="s")
    return pl.kernel(
        _sc_body,
        mesh=mesh,
        out_type=(
            jax.ShapeDtypeStruct((B, LANES), jnp.float32),
            jax.ShapeDtypeStruct((HV_TOTAL, LANES), jnp.float32),
        ),
        scratch_types=[
            pltpu.VMEM((CAT_PW,), jnp.int32),
            pltpu.VMEM((HV_CHUNKS_PW, HV_CHUNK), jnp.int32),
            pltpu.VMEM((CAT_PW, LANES), jnp.float32),
            pltpu.VMEM((HV_PW, LANES), jnp.float32),
            pltpu.SemaphoreType.DMA,
        ],
    )(cat_sup_ix, hv_sup_ix, cat_sup, hv_sup)


def _select_sub(rows, sub, width):
    out = jnp.where(sub == 0, rows[:, 0:width], 0.0)
    for p in range(1, LANES // width):
        out = out + jnp.where(sub == p, rows[:, p * width:(p + 1) * width],
                              0.0)
    return out


def _dedup_weighted_sum(ix, hv, base):
    sub = jnp.bitwise_and(jnp.right_shift(ix, 7), HV_PER_SUP - 1)
    acc = _select_sub(hv[base], sub[:, 0:1], SEM)
    for j in range(1, K):
        dup = (ix[:, 0:1] == ix[:, j:j + 1]).astype(jnp.float32)
        for i in range(1, j):
            dup = jnp.maximum(
                dup, (ix[:, i:i + 1] == ix[:, j:j + 1]).astype(jnp.float32))
        acc = acc + (1.0 - dup) * _select_sub(hv[base + j], sub[:, j:j + 1],
                                              SEM)
    return acc


def _tc_body(cat_rows_ref, hv_rows_ref, cat_ix_ref, hvb_ix_ref, hvf_ix_ref,
             hvb_top_ref, hvf_top_ref, d_onehot_ref,
             w1t_ref, b1_ref, w2t_ref, b2_ref, out_ref):
    hv = hv_rows_ref[...]
    hvb_e = _dedup_weighted_sum(hvb_ix_ref[...], hv, 0) + hvb_top_ref[...]
    hvf_e = _dedup_weighted_sum(hvf_ix_ref[...], hv, K) + hvf_top_ref[...]
    cat_sub = jnp.bitwise_and(jnp.right_shift(cat_ix_ref[...], 7),
                              CAT_PER_SUP - 1)
    cat_e = _select_sub(cat_rows_ref[...], cat_sub, SYN)
    x = jnp.concatenate([cat_e, hvb_e, hvf_e, d_onehot_ref[...]], axis=1)
    h = jnp.maximum(
        jnp.dot(x, w1t_ref[...], preferred_element_type=jnp.float32)
        + b1_ref[...], 0.0)
    o = jnp.dot(h, w2t_ref[...], preferred_element_type=jnp.float32)
    o = o + b2_ref[...]
    m = jnp.max(o, axis=1, keepdims=True)
    s = o - m
    out_ref[...] = s - jnp.log(jnp.sum(jnp.exp(s), axis=1, keepdims=True))


def _tc_mlp(cat_rows, hv_rows, cat_ix, hvb_ix, hvf_ix, hvb_top, hvf_top,
            d_onehot, w1t, b1, w2t, b2):
    out_dim = w2t.shape[1]
    nb = 2
    bb = B // nb
    full = lambda shape: pl.BlockSpec(shape, lambda g: tuple(0 for _ in shape))
    row = lambda cols: pl.BlockSpec((bb, cols), lambda g: (g, 0))
    return pl.pallas_call(
        _tc_body,
        grid=(nb,),
        in_specs=[
            row(LANES),
            pl.BlockSpec((2 * K, bb, LANES), lambda g: (0, g, 0)),
            row(1), row(K), row(K), row(SEM), row(SEM), row(7),
            full(w1t.shape), full(b1.shape), full(w2t.shape), full(b2.shape),
        ],
        out_specs=row(out_dim),
        out_shape=jax.ShapeDtypeStruct((B, out_dim), jnp.float32),
    )(cat_rows, hv_rows, cat_ix, hvb_ix, hvf_ix, hvb_top, hvf_top,
      d_onehot, w1t, b1, w2t, b2)


def kernel(d_onehot, cat_b_ix, hvb_ix, hvf_ix, hvb_top, hvf_top, use_gpu,
           cat_emb, hvec_emb, fc1_w, fc1_b, fc2_w, fc2_b):
    cat_ix = cat_b_ix.astype(jnp.int32)
    hvb_i = hvb_ix.astype(jnp.int32)
    hvf_i = hvf_ix.astype(jnp.int32)
    hv_flat = jnp.concatenate([hvb_i.T.reshape(-1), hvf_i.T.reshape(-1)])
    hv_sup_ix = (
        ((hv_flat >> 9) << 7) + (hv_flat & 127)
    ).reshape(HV_TOTAL // HV_CHUNK, HV_CHUNK)
    cat_sup_ix = ((cat_ix >> 8) << 7) + (cat_ix & 127)
    cat_sup = _compact(cat_emb.T, 10240)
    hv_sup = _compact(hvec_emb.T, 16384)
    cat_rows, hv_rows = _sc_gather(cat_sup_ix, hv_sup_ix, cat_sup, hv_sup)
    return _tc_mlp(cat_rows, hv_rows.reshape(2 * K, B, LANES),
                   cat_ix.reshape(B, 1), hvb_i, hvf_i,
                   hvb_top, hvf_top, d_onehot,
                   fc1_w.T, fc1_b.reshape(1, -1),
                   fc2_w.T, fc2_b.reshape(1, -1))


# speedup vs baseline: 1.1287x; 1.0028x over previous
import jax
import jax.numpy as jnp
from jax import lax
from jax.experimental import pallas as pl
from jax.experimental.pallas import tpu as pltpu
from jax.experimental.pallas import tpu_sc as plsc

B = 1024
K = 5
SYN = 64
SEM = 32
LANES = 128
HV_PER_SUP = LANES // SEM
CAT_PER_SUP = LANES // SYN
NC = 2
NS = 16
NW = NC * NS
CAT_PW = B // NW
HV_TOTAL = 2 * B * K
HV_CHUNK = 80
HV_CHUNKS_PW = HV_TOTAL // (NW * HV_CHUNK)
HV_PW = HV_CHUNK * HV_CHUNKS_PW


def _compact_body(in_ref, out_ref):
    x = in_ref[...]
    d = x.shape[0]
    r = LANES // d
    slabs = []
    for s in range(x.shape[1] // (r * LANES)):
        v = jnp.concatenate(
            [x[:, (s * r + m) * LANES:(s * r + m + 1) * LANES]
             for m in range(r)], axis=0)
        slabs.append(jnp.transpose(v))
    out_ref[...] = jnp.concatenate(slabs, axis=0)


def _compact(table_t, blk_cols):
    d, v = table_t.shape
    blk_rows = d * blk_cols // LANES
    n_blocks = -(-v // blk_cols)
    out_rows = n_blocks * blk_rows
    return pl.pallas_call(
        _compact_body,
        grid=(n_blocks,),
        in_specs=[pl.BlockSpec((d, blk_cols), lambda c: (0, c))],
        out_specs=pl.BlockSpec((blk_rows, LANES), lambda c: (c, 0)),
        out_shape=jax.ShapeDtypeStruct((out_rows, LANES), jnp.float32),
    )(table_t)


def _sc_body(cat_ix_hbm, hv_ix_hbm, cat_sup_hbm, hv_sup_hbm,
             cat_out_hbm, hv_out_hbm,
             cat_idx_v, hv_idx_v, cat_rows_v, hv_rows_v, sem):
    wid = lax.axis_index("s") * NC + lax.axis_index("c")
    pltpu.sync_copy(cat_ix_hbm.at[pl.ds(wid * CAT_PW, CAT_PW)], cat_idx_v)
    pltpu.sync_copy(hv_ix_hbm.at[pl.ds(wid * HV_CHUNKS_PW, HV_CHUNKS_PW)],
                    hv_idx_v)
    copies = [pltpu.async_copy(cat_sup_hbm.at[cat_idx_v], cat_rows_v, sem)]
    for j in range(HV_CHUNKS_PW):
        copies.append(pltpu.async_copy(
            hv_sup_hbm.at[hv_idx_v.at[j]],
            hv_rows_v.at[pl.ds(j * HV_CHUNK, HV_CHUNK)], sem))
    for c in copies:
        c.wait()
    pltpu.sync_copy(cat_rows_v, cat_out_hbm.at[pl.ds(wid * CAT_PW, CAT_PW)])
    pltpu.sync_copy(hv_rows_v, hv_out_hbm.at[pl.ds(wid * HV_PW, HV_PW)])


def _sc_gather(cat_sup_ix, hv_sup_ix, cat_sup, hv_sup):
    mesh = plsc.VectorSubcoreMesh(core_axis_name="c", subcore_axis_name="s")
    return pl.kernel(
        _sc_body,
        mesh=mesh,
        out_type=(
            jax.ShapeDtypeStruct((B, LANES), jnp.float32),
            jax.ShapeDtypeStruct((HV_TOTAL, LANES), jnp.float32),
        ),
        scratch_types=[
            pltpu.VMEM((CAT_PW,), jnp.int32),
            pltpu.VMEM((HV_CHUNKS_PW, HV_CHUNK), jnp.int32),
            pltpu.VMEM((CAT_PW, LANES), jnp.float32),
            pltpu.VMEM((HV_PW, LANES), jnp.float32),
            pltpu.SemaphoreType.DMA,
        ],
    )(cat_sup_ix, hv_sup_ix, cat_sup, hv_sup)


def _select_sub(rows, sub, width):
    out = jnp.where(sub == 0, rows[:, 0:width], 0.0)
    for p in range(1, LANES // width):
        out = out + jnp.where(sub == p, rows[:, p * width:(p + 1) * width],
                              0.0)
    return out


def _dedup_weighted_sum(ix, hv, base):
    sub = jnp.bitwise_and(jnp.right_shift(ix, 7), HV_PER_SUP - 1)
    acc = _select_sub(hv[base], sub[:, 0:1], SEM)
    for j in range(1, K):
        dup = (ix[:, 0:1] == ix[:, j:j + 1]).astype(jnp.float32)
        for i in range(1, j):
            dup = jnp.maximum(
                dup, (ix[:, i:i + 1] == ix[:, j:j + 1]).astype(jnp.float32))
        acc = acc + (1.0 - dup) * _select_sub(hv[base + j], sub[:, j:j + 1],
                                              SEM)
    return acc


def _tc_body(cat_rows_ref, hv_rows_ref, cat_ix_ref, hvb_ix_ref, hvf_ix_ref,
             hvb_top_ref, hvf_top_ref, d_onehot_ref,
             w1t_ref, b1_ref, w2t_ref, b2_ref, out_ref):
    hv = hv_rows_ref[...]
    hvb_e = _dedup_weighted_sum(hvb_ix_ref[...], hv, 0) + hvb_top_ref[...]
    hvf_e = _dedup_weighted_sum(hvf_ix_ref[...], hv, K) + hvf_top_ref[...]
    cat_sub = jnp.bitwise_and(jnp.right_shift(cat_ix_ref[...], 7),
                              CAT_PER_SUP - 1)
    cat_e = _select_sub(cat_rows_ref[...], cat_sub, SYN)
    x = jnp.concatenate([cat_e, hvb_e, hvf_e, d_onehot_ref[...]], axis=1)
    h = jnp.maximum(
        jnp.dot(x, w1t_ref[...], preferred_element_type=jnp.float32)
        + b1_ref[...], 0.0)
    o = jnp.dot(h, w2t_ref[...], preferred_element_type=jnp.float32)
    o = o + b2_ref[...]
    m = jnp.max(o, axis=1, keepdims=True)
    s = o - m
    out_ref[...] = s - jnp.log(jnp.sum(jnp.exp(s), axis=1, keepdims=True))


def _tc_mlp(cat_rows, hv_rows, cat_ix, hvb_ix, hvf_ix, hvb_top, hvf_top,
            d_onehot, w1t, b1, w2t, b2):
    out_dim = w2t.shape[1]
    nb = 2
    bb = B // nb
    full = lambda shape: pl.BlockSpec(shape, lambda g: tuple(0 for _ in shape))
    row = lambda cols: pl.BlockSpec((bb, cols), lambda g: (g, 0))
    return pl.pallas_call(
        _tc_body,
        grid=(nb,),
        in_specs=[
            row(LANES),
            pl.BlockSpec((2 * K, bb, LANES), lambda g: (0, g, 0)),
            row(1), row(K), row(K), row(SEM), row(SEM), row(7),
            full(w1t.shape), full(b1.shape), full(w2t.shape), full(b2.shape),
        ],
        out_specs=row(out_dim),
        out_shape=jax.ShapeDtypeStruct((B, out_dim), jnp.float32),
    )(cat_rows, hv_rows, cat_ix, hvb_ix, hvf_ix, hvb_top, hvf_top,
      d_onehot, w1t, b1, w2t, b2)


def kernel(d_onehot, cat_b_ix, hvb_ix, hvf_ix, hvb_top, hvf_top, use_gpu,
           cat_emb, hvec_emb, fc1_w, fc1_b, fc2_w, fc2_b):
    cat_ix = cat_b_ix.astype(jnp.int32)
    hvb_i = hvb_ix.astype(jnp.int32)
    hvf_i = hvf_ix.astype(jnp.int32)
    hv_flat = jnp.concatenate([hvb_i.T.reshape(-1), hvf_i.T.reshape(-1)])
    hv_sup_ix = (
        ((hv_flat >> 9) << 7) + (hv_flat & 127)
    ).reshape(HV_TOTAL // HV_CHUNK, HV_CHUNK)
    cat_sup_ix = ((cat_ix >> 8) << 7) + (cat_ix & 127)
    cat_sup = _compact(cat_emb.T, 10240)
    hv_sup = _compact(hvec_emb.T, 20480)
    cat_rows, hv_rows = _sc_gather(cat_sup_ix, hv_sup_ix, cat_sup, hv_sup)
    return _tc_mlp(cat_rows, hv_rows.reshape(2 * K, B, LANES),
                   cat_ix.reshape(B, 1), hvb_i, hvf_i,
                   hvb_top, hvf_top, d_onehot,
                   fc1_w.T, fc1_b.reshape(1, -1),
                   fc2_w.T, fc2_b.reshape(1, -1))
